# Initial kernel scaffold; baseline (speedup 1.0000x reference)
import jax
import jax.numpy as jnp
from jax.experimental import pallas as pl


def kernel(prefix, table):
    raise NotImplementedError("write your pallas kernel here")


# speedup vs baseline: 1.6364x; 1.6364x over previous
import functools

import jax
import jax.numpy as jnp
from jax import lax
from jax.experimental import pallas as pl
from jax.experimental.pallas import tpu as pltpu
from jax.experimental.pallas import tpu_sc as plsc

PRE_SEQ_LEN = 128
NUM_LAYERS = 12
HIDDEN = 768
EMB_DIM = NUM_LAYERS * 2 * HIDDEN
BATCH = 32
PREFIX_LEN = 128

NC = 2
NS = 16
NW = NC * NS
B_TOTAL = BATCH * PREFIX_LEN
B_PER_W = B_TOTAL // NW
G = 4
NCHUNK = B_PER_W // G


def _make_gather():
    mesh = plsc.VectorSubcoreMesh(core_axis_name="c", subcore_axis_name="s")

    @functools.partial(
        pl.kernel,
        mesh=mesh,
        out_type=jax.ShapeDtypeStruct((B_TOTAL, EMB_DIM), jnp.float32),
        scratch_types=[
            pltpu.VMEM((NCHUNK, G), jnp.int32),
            pltpu.VMEM((G, EMB_DIM), jnp.float32),
            pltpu.SemaphoreType.DMA,
        ],
    )
    def gather_kernel(idx_hbm, table_hbm, out_hbm, idx_v, rows_v, sem):
        wid = lax.axis_index("s") * NC + lax.axis_index("c")
        pltpu.sync_copy(idx_hbm.at[wid], idx_v)
        base = wid * B_PER_W

        def body(c, carry):
            pltpu.async_copy(table_hbm.at[idx_v.at[c]], rows_v, sem).wait()
            pltpu.sync_copy(rows_v, out_hbm.at[pl.ds(base + c * G, G)])
            return carry

        lax.fori_loop(0, NCHUNK, body, 0)

    return gather_kernel


_gather = _make_gather()


@jax.jit
def kernel(prefix, table):
    idx = prefix.astype(jnp.int32).reshape(NW, NCHUNK, G)
    out = _gather(idx, table)
    return out.reshape(BATCH, PREFIX_LEN, EMB_DIM)


# speedup vs baseline: 1.7721x; 1.0829x over previous
import functools

import jax
import jax.numpy as jnp
from jax import lax
from jax.experimental import pallas as pl
from jax.experimental.pallas import tpu as pltpu
from jax.experimental.pallas import tpu_sc as plsc

PRE_SEQ_LEN = 128
NUM_LAYERS = 12
HIDDEN = 768
EMB_DIM = NUM_LAYERS * 2 * HIDDEN
BATCH = 32
PREFIX_LEN = 128

NC = 2
NS = 16
NW = NC * NS
B_TOTAL = BATCH * PREFIX_LEN
B_PER_W = B_TOTAL // NW
G = 1
NCHUNK = B_PER_W // G
NBUF = 4


def _make_gather():
    mesh = plsc.VectorSubcoreMesh(core_axis_name="c", subcore_axis_name="s")

    @functools.partial(
        pl.kernel,
        mesh=mesh,
        out_type=jax.ShapeDtypeStruct((B_TOTAL, EMB_DIM), jnp.float32),
        scratch_types=[
            pltpu.VMEM((NCHUNK, G), jnp.int32),
            pltpu.VMEM((NBUF, G, EMB_DIM), jnp.float32),
            pltpu.SemaphoreType.DMA,
            pltpu.SemaphoreType.DMA,
        ],
    )
    def gather_kernel(idx_hbm, table_hbm, out_hbm, idx_v, rows_v, gsem, ssem):
        wid = lax.axis_index("s") * NC + lax.axis_index("c")
        pltpu.sync_copy(idx_hbm.at[wid], idx_v)
        base = wid * B_PER_W

        def g_start(c, b):
            pltpu.async_copy(table_hbm.at[idx_v.at[c]], rows_v.at[b], gsem)

        def g_wait(c, b):
            pltpu.make_async_copy(
                table_hbm.at[idx_v.at[c]], rows_v.at[b], gsem).wait()

        def s_start(c, b):
            pltpu.async_copy(
                rows_v.at[b], out_hbm.at[pl.ds(base + c * G, G)], ssem)

        def s_wait(c, b):
            pltpu.make_async_copy(
                rows_v.at[b], out_hbm.at[pl.ds(base + c * G, G)], ssem).wait()

        for b in range(NBUF):
            g_start(b, b)

        def body(c, carry):
            b = lax.rem(c, NBUF)
            g_wait(c, b)
            s_start(c, b)

            @pl.when(c + NBUF < NCHUNK)
            def _():
                s_wait(c, b)
                g_start(c + NBUF, b)

            return carry

        lax.fori_loop(0, NCHUNK, body, 0)

        for b in range(NBUF):
            s_wait(b, b)

    return gather_kernel


_gather = _make_gather()


@jax.jit
def kernel(prefix, table):
    idx = prefix.astype(jnp.int32).reshape(NW, NCHUNK, G)
    out = _gather(idx, table)
    return out.reshape(BATCH, PREFIX_LEN, EMB_DIM)
